# Initial kernel scaffold; baseline (speedup 1.0000x reference)
import jax
import jax.numpy as jnp
from jax.experimental import pallas as pl


def kernel(atomic_numbers, embedding_table):
    raise NotImplementedError("write your pallas kernel here")


# speedup vs baseline: 2.9669x; 2.9669x over previous
import jax
import jax.numpy as jnp
from jax.experimental import pallas as pl
from jax.experimental.pallas import tpu as pltpu
from jax.experimental.pallas import tpu_sc as plsc

WINDOW = 128


def kernel(atomic_numbers, embedding_table):
    B, S = atomic_numbers.shape
    V, D = embedding_table.shape
    n = B * S
    idx = atomic_numbers.reshape(1, n)

    mesh = plsc.VectorSubcoreMesh(core_axis_name="c", subcore_axis_name="s")

    @pl.kernel(
        out_type=jax.ShapeDtypeStruct((n, D), embedding_table.dtype),
        mesh=mesh,
    )
    def gather_kernel(table_hbm, idx_hbm, out_hbm):
        def body(i_vmem, o_vmem):
            pltpu.sync_copy(table_hbm.at[i_vmem.at[0]], o_vmem)

        pltpu.emit_pipeline(
            body,
            grid=(n // WINDOW,),
            in_specs=[pl.BlockSpec((1, WINDOW), index_map=lambda i: (0, i))],
            out_specs=[pl.BlockSpec((WINDOW, D), index_map=lambda i: (i, 0))],
            core_axis_name=("c", "s"),
            dimension_semantics=(pltpu.PARALLEL,),
        )(idx_hbm, out_hbm)

    out = gather_kernel(embedding_table, idx)
    return out.reshape(B, S, D)


# speedup vs baseline: 3.0112x; 1.0149x over previous
import jax
import jax.numpy as jnp
from jax.experimental import pallas as pl
from jax.experimental.pallas import tpu as pltpu
from jax.experimental.pallas import tpu_sc as plsc

WINDOW = 256


def kernel(atomic_numbers, embedding_table):
    B, S = atomic_numbers.shape
    V, D = embedding_table.shape
    n = B * S
    idx = atomic_numbers.reshape(1, n)

    mesh = plsc.VectorSubcoreMesh(core_axis_name="c", subcore_axis_name="s")

    @pl.kernel(
        out_type=jax.ShapeDtypeStruct((n, D), embedding_table.dtype),
        mesh=mesh,
    )
    def gather_kernel(table_hbm, idx_hbm, out_hbm):
        def body(i_vmem, o_vmem):
            pltpu.sync_copy(table_hbm.at[i_vmem.at[0]], o_vmem)

        pltpu.emit_pipeline(
            body,
            grid=(n // WINDOW,),
            in_specs=[pl.BlockSpec((1, WINDOW), index_map=lambda i: (0, i))],
            out_specs=[pl.BlockSpec((WINDOW, D), index_map=lambda i: (i, 0))],
            core_axis_name=("c", "s"),
            dimension_semantics=(pltpu.PARALLEL,),
        )(idx_hbm, out_hbm)

    out = gather_kernel(embedding_table, idx)
    return out.reshape(B, S, D)


# speedup vs baseline: 4.0402x; 1.3417x over previous
import jax
import jax.numpy as jnp
from jax import lax
from jax.experimental import pallas as pl
from jax.experimental.pallas import tpu as pltpu
from jax.experimental.pallas import tpu_sc as plsc

WINDOW = 128


def kernel(atomic_numbers, embedding_table):
    B, S = atomic_numbers.shape
    V, D = embedding_table.shape
    n = B * S
    idx = atomic_numbers.reshape(1, n)

    mesh = plsc.VectorSubcoreMesh(core_axis_name="c", subcore_axis_name="s")

    @pl.kernel(
        out_type=jax.ShapeDtypeStruct((n, D), embedding_table.dtype),
        mesh=mesh,
        scratch_types=[pltpu.VMEM_SHARED((V, D), embedding_table.dtype)],
    )
    def gather_kernel(table_hbm, idx_hbm, out_hbm, table_spmem):
        @pl.when(lax.axis_index("s") == 0)
        def _():
            pltpu.sync_copy(table_hbm, table_spmem)

        plsc.subcore_barrier()

        def body(i_vmem, o_vmem):
            pltpu.sync_copy(table_spmem.at[i_vmem.at[0]], o_vmem)

        pltpu.emit_pipeline(
            body,
            grid=(n // WINDOW,),
            in_specs=[pl.BlockSpec((1, WINDOW), index_map=lambda i: (0, i))],
            out_specs=[pl.BlockSpec((WINDOW, D), index_map=lambda i: (i, 0))],
            core_axis_name=("c", "s"),
            dimension_semantics=(pltpu.PARALLEL,),
        )(idx_hbm, out_hbm)

    out = gather_kernel(embedding_table, idx)
    return out.reshape(B, S, D)


# speedup vs baseline: 4.0814x; 1.0102x over previous
import jax
import jax.numpy as jnp
from jax import lax
from jax.experimental import pallas as pl
from jax.experimental.pallas import tpu as pltpu
from jax.experimental.pallas import tpu_sc as plsc

WINDOW = 128


def kernel(atomic_numbers, embedding_table):
    B, S = atomic_numbers.shape
    V, D = embedding_table.shape
    n = B * S
    idx = atomic_numbers.reshape(1, n)

    mesh = plsc.VectorSubcoreMesh(core_axis_name="c", subcore_axis_name="s")

    K = 2
    STEP = K * WINDOW

    @pl.kernel(
        out_type=jax.ShapeDtypeStruct((n, D), embedding_table.dtype),
        mesh=mesh,
        scratch_types=[
            pltpu.VMEM_SHARED((V, D), embedding_table.dtype),
            pltpu.SemaphoreType.DMA,
        ],
    )
    def gather_kernel(table_hbm, idx_hbm, out_hbm, table_spmem, sem):
        @pl.when(lax.axis_index("s") == 0)
        def _():
            pltpu.sync_copy(table_hbm, table_spmem)

        plsc.subcore_barrier()

        def body(i_vmem, o_vmem):
            copies = [
                pltpu.async_copy(
                    table_spmem.at[i_vmem.at[0, pl.ds(k * WINDOW, WINDOW)]],
                    o_vmem.at[pl.ds(k * WINDOW, WINDOW)],
                    sem,
                )
                for k in range(K)
            ]
            for c in copies:
                c.wait()

        pltpu.emit_pipeline(
            body,
            grid=(n // STEP,),
            in_specs=[pl.BlockSpec((1, STEP), index_map=lambda i: (0, i))],
            out_specs=[pl.BlockSpec((STEP, D), index_map=lambda i: (i, 0))],
            core_axis_name=("c", "s"),
            dimension_semantics=(pltpu.PARALLEL,),
        )(idx_hbm, out_hbm)

    out = gather_kernel(embedding_table, idx)
    return out.reshape(B, S, D)


# speedup vs baseline: 7.7824x; 1.9068x over previous
import jax
import jax.numpy as jnp
from jax import lax
from jax.experimental import pallas as pl
from jax.experimental.pallas import tpu as pltpu
from jax.experimental.pallas import tpu_sc as plsc

WINDOW = 128


def kernel(atomic_numbers, embedding_table):
    B, S = atomic_numbers.shape
    V, D = embedding_table.shape

    mesh = plsc.VectorSubcoreMesh(core_axis_name="c", subcore_axis_name="s")

    BLK_B = 2

    @pl.kernel(
        out_type=jax.ShapeDtypeStruct((B, S, D), embedding_table.dtype),
        mesh=mesh,
        scratch_types=[
            pltpu.VMEM_SHARED((V, D), embedding_table.dtype),
            pltpu.SemaphoreType.DMA,
        ],
    )
    def gather_kernel(table_hbm, idx_hbm, out_hbm, table_spmem, sem):
        @pl.when(lax.axis_index("s") == 0)
        def _():
            pltpu.sync_copy(table_hbm, table_spmem)

        plsc.subcore_barrier()

        def body(i_vmem, o_vmem):
            copies = [
                pltpu.async_copy(
                    table_spmem.at[i_vmem.at[k]],
                    o_vmem.at[k],
                    sem,
                )
                for k in range(BLK_B)
            ]
            for c in copies:
                c.wait()

        pltpu.emit_pipeline(
            body,
            grid=(B // BLK_B,),
            in_specs=[pl.BlockSpec((BLK_B, S), index_map=lambda i: (i, 0))],
            out_specs=[
                pl.BlockSpec((BLK_B, S, D), index_map=lambda i: (i, 0, 0))
            ],
            core_axis_name=("c", "s"),
            dimension_semantics=(pltpu.PARALLEL,),
        )(idx_hbm, out_hbm)

    return gather_kernel(embedding_table, atomic_numbers)


# speedup vs baseline: 7.8012x; 1.0024x over previous
import jax
import jax.numpy as jnp
from jax import lax
from jax.experimental import pallas as pl
from jax.experimental.pallas import tpu as pltpu
from jax.experimental.pallas import tpu_sc as plsc

WINDOW = 128


def kernel(atomic_numbers, embedding_table):
    B, S = atomic_numbers.shape
    V, D = embedding_table.shape

    mesh = plsc.VectorSubcoreMesh(core_axis_name="c", subcore_axis_name="s")

    BLK_B = 4

    @pl.kernel(
        out_type=jax.ShapeDtypeStruct((B, S, D), embedding_table.dtype),
        mesh=mesh,
        scratch_types=[
            pltpu.VMEM_SHARED((V, D), embedding_table.dtype),
            pltpu.SemaphoreType.DMA,
        ],
    )
    def gather_kernel(table_hbm, idx_hbm, out_hbm, table_spmem, sem):
        @pl.when(lax.axis_index("s") == 0)
        def _():
            pltpu.sync_copy(table_hbm, table_spmem)

        plsc.subcore_barrier()

        def body(i_vmem, o_vmem):
            copies = [
                pltpu.async_copy(
                    table_spmem.at[i_vmem.at[k]],
                    o_vmem.at[k],
                    sem,
                )
                for k in range(BLK_B)
            ]
            for c in copies:
                c.wait()

        pltpu.emit_pipeline(
            body,
            grid=(B // BLK_B,),
            in_specs=[pl.BlockSpec((BLK_B, S), index_map=lambda i: (i, 0))],
            out_specs=[
                pl.BlockSpec((BLK_B, S, D), index_map=lambda i: (i, 0, 0))
            ],
            core_axis_name=("c", "s"),
            dimension_semantics=(pltpu.PARALLEL,),
        )(idx_hbm, out_hbm)

    return gather_kernel(embedding_table, atomic_numbers)
